# baseline (speedup 1.0000x reference)
import functools

import jax
import jax.numpy as jnp
from jax import lax
from jax.experimental import pallas as pl
from jax.experimental.pallas import tpu as pltpu
from jax.experimental.pallas import tpu_sc as plsc

NUM_EMB = 1_000_000
DIM = 32
WORDS = DIM // 2
ROWS = 16384 * 50
NW = 32
ROWS_W = ROWS // NW
CHUNK = 128
NCH = ROWS_W // CHUNK
NBUF = 8
NGRP = NCH // NBUF


def _emb_body(idx_hbm, table_hbm, out_hbm, idx_v, rows_v, sem_g, sem_w):
    c = lax.axis_index("c")
    s = lax.axis_index("s")
    w = s * 2 + c
    base = w * ROWS_W

    pltpu.sync_copy(idx_hbm.at[w], idx_v)

    def group(g, carry):
        ch0 = g * NBUF
        handles = []
        for r in range(NBUF):
            h = pltpu.async_copy(
                table_hbm.at[idx_v.at[ch0 + r]], rows_v.at[r], sem_g)
            handles.append(h)
        for h in handles:
            h.wait()
        wbs = []
        for r in range(NBUF):
            h = pltpu.async_copy(
                rows_v.at[r],
                out_hbm.at[pl.ds(base + (ch0 + r) * CHUNK, CHUNK)],
                sem_w)
            wbs.append(h)
        for h in wbs:
            h.wait()
        return carry

    lax.fori_loop(0, NGRP, group, 0)


@jax.jit
def _emb_call(idx, table_i32):
    mesh = plsc.VectorSubcoreMesh(core_axis_name="c", subcore_axis_name="s")
    f = pl.kernel(
        _emb_body,
        out_type=jax.ShapeDtypeStruct((ROWS, WORDS), jnp.int32),
        mesh=mesh,
        scratch_types=[
            pltpu.VMEM((NCH, CHUNK), jnp.int32),
            pltpu.VMEM((NBUF, CHUNK, WORDS), jnp.int32),
            pltpu.SemaphoreType.DMA,
            pltpu.SemaphoreType.DMA,
        ],
        compiler_params=pltpu.CompilerParams(use_tc_tiling_on_sc=False),
    )
    return f(idx, table_i32)


def kernel(input, weight):
    idx = input.reshape(NW, NCH, CHUNK).astype(jnp.int32)
    table_i32 = jax.lax.bitcast_convert_type(
        weight.reshape(NUM_EMB, WORDS, 2), jnp.int32)
    out = _emb_call(idx, table_i32)
    out_bf = jax.lax.bitcast_convert_type(out, jnp.bfloat16)
    return out_bf.reshape(input.shape[0], input.shape[1], DIM)


# speedup vs baseline: 1.7755x; 1.7755x over previous
import jax
import jax.numpy as jnp
from jax import lax
from jax.experimental import pallas as pl
from jax.experimental.pallas import tpu as pltpu
from jax.experimental.pallas import tpu_sc as plsc

NUM_EMB = 1_000_000
DIM = 32
WORDS = DIM // 2
PAIRS = NUM_EMB // 2
BATCH = 16384
HIST = 50
NW = 32
ABLK = 640
NABLK = 25
PAIRS_W = NABLK * ABLK
PAD_PLANE = PAIRS_W * NW
BTILES = BATCH // 128
BT_W = BTILES // NW
NBLK = HIST * BT_W
NB = 4

_SC_PARAMS = pltpu.CompilerParams(
    use_tc_tiling_on_sc=False, needs_layout_passes=False)


def _worker_id():
    return lax.axis_index("s") * 2 + lax.axis_index("c")


def _format_body(wt_hbm, tr_hbm, bufE, bufO, trbuf, sin, sout):
    w = _worker_id()
    p_base = w * PAIRS_W
    wsrc = wt_hbm.bitcast(jnp.int32)

    iot = lax.iota(jnp.int32, 16)

    def stage(b, slot):
        p0 = p_base + b * ABLK
        for r in range(WORDS):
            pltpu.async_copy(
                wsrc.at[r, pl.ds(p0, ABLK)], bufE.at[slot, r], sin[slot])
            pltpu.async_copy(
                wsrc.at[r, pl.ds(PAD_PLANE + p0, ABLK)], bufO.at[slot, r],
                sin[slot])

    def drain_in(slot):
        d = pltpu.make_async_copy(
            wsrc.at[0, pl.ds(0, ABLK)], bufE.at[slot, 0], sin[slot])
        for _ in range(2 * WORDS):
            d.wait()

    def drain_out(slot):
        pltpu.make_async_copy(
            trbuf.at[slot], tr_hbm.at[pl.ds(0, 2 * ABLK)], sout[slot]).wait()

    def compute(slot):
        def grp(g, carry):
            for jj in range(16):
                j = g * 16 + jj
                colv = jnp.full((16,), 0, jnp.int32) + j
                va = plsc.load_gather(bufE.at[slot], [iot, colv])
                vb = plsc.load_gather(bufO.at[slot], [iot, colv])
                even = jnp.bitwise_or(
                    jnp.left_shift(vb, 16), jnp.bitwise_and(va, 0xFFFF))
                odd = jnp.bitwise_or(
                    lax.shift_right_logical(va, 16),
                    jnp.bitwise_and(vb, jnp.int32(-65536)))
                trbuf[slot, 2 * j, :] = even
                trbuf[slot, 2 * j + 1, :] = odd
            return carry
        lax.fori_loop(0, ABLK // 16, grp, 0)

    def writeback(b, slot):
        pltpu.async_copy(
            trbuf.at[slot],
            tr_hbm.at[pl.ds(2 * (p_base + b * ABLK), 2 * ABLK)],
            sout[slot])

    stage(jnp.int32(0), 0)
    stage(jnp.int32(1), 1)

    def two_blocks(i, carry):
        for slot in range(2):
            b = 2 * i + slot
            drain_in(slot)

            @pl.when(i > 0)
            def _():
                drain_out(slot)

            compute(slot)
            writeback(b, slot)

            @pl.when(b + 2 < NABLK)
            def _():
                stage(b + 2, slot)
        return carry

    lax.fori_loop(0, NABLK // 2, two_blocks, 0)
    drain_in(0)
    drain_out(0)
    compute(0)
    writeback(jnp.int32(NABLK - 1), 0)
    drain_out(0)
    drain_out(1)


def _gather_body(idx_hbm, table_hbm, out_hbm, idx_v, rows_v, tb, sems):
    (sg0, sg1, sg2, sg3, sw0, sw1) = sems
    sgs = (sg0, sg1, sg2, sg3)
    sws = (sw0, sw1)
    w = _worker_id()

    pltpu.sync_copy(idx_hbm.at[:, pl.ds(BT_W * w, BT_W), :], idx_v)

    iot = lax.iota(jnp.int32, 16)

    def fire(k, slot):
        h = k // BT_W
        bb = lax.rem(k, BT_W)
        return pltpu.async_copy(
            table_hbm.at[idx_v.at[h, bb]], rows_v.at[slot], sgs[slot])

    def transpose_block(slot, tslot):
        for t in range(2):
            for r in range(8):
                colv = jnp.full((16,), 8 * t + r, jnp.int32)
                for g in range(8):
                    vec = plsc.load_gather(
                        rows_v.at[slot], [iot + 16 * g, colv])
                    tb[tslot, t, r, pl.ds(16 * g, 16)] = vec

    def writeback(k, tslot):
        h = k // BT_W
        bb = lax.rem(k, BT_W)
        bg = BT_W * w + bb
        pltpu.async_copy(tb.at[tslot, 0], out_hbm.at[h, 0, bg], sws[tslot])
        pltpu.async_copy(tb.at[tslot, 1], out_hbm.at[h, 1, bg], sws[tslot])

    def drain_gather(slot):
        pltpu.make_async_copy(
            table_hbm.at[idx_v.at[0, 0]], rows_v.at[slot], sgs[slot]).wait()

    def drain_wb(tslot):
        d = pltpu.make_async_copy(tb.at[tslot, 0], out_hbm.at[0, 0, 0], sws[tslot])
        d.wait()
        d.wait()

    for k in range(NB):
        fire(jnp.int32(k), k)

    def group(gi, carry):
        k0 = gi * NB
        for slot in range(NB):
            k = k0 + slot
            drain_gather(slot)
            tslot = slot % 2
            if slot < 2:
                @pl.when(gi > 0)
                def _():
                    drain_wb(tslot)
            else:
                drain_wb(tslot)
            transpose_block(slot, tslot)
            writeback(k, tslot)

            @pl.when(k + NB < NBLK)
            def _():
                fire(k + NB, slot)
        return carry

    lax.fori_loop(0, NBLK // NB, group, 0)
    drain_wb(0)
    drain_wb(1)


@jax.jit
def _emb_call(idx3, wt):
    mesh = plsc.VectorSubcoreMesh(core_axis_name="c", subcore_axis_name="s")
    fa = pl.kernel(
        _format_body,
        out_type=jax.ShapeDtypeStruct((2 * PAD_PLANE, WORDS), jnp.int32),
        mesh=mesh,
        scratch_types=[
            pltpu.VMEM((2, WORDS, ABLK), jnp.int32),
            pltpu.VMEM((2, WORDS, ABLK), jnp.int32),
            pltpu.VMEM((2, 2 * ABLK, WORDS), jnp.int32),
            (pltpu.SemaphoreType.DMA,) * 2,
            (pltpu.SemaphoreType.DMA,) * 2,
        ],
        compiler_params=_SC_PARAMS,
    )
    table_i32 = fa(wt)
    fb = pl.kernel(
        _gather_body,
        out_type=jax.ShapeDtypeStruct((HIST, 2, BTILES, 8, 128), jnp.int32),
        mesh=mesh,
        scratch_types=[
            pltpu.VMEM((HIST, BT_W, 128), jnp.int32),
            pltpu.VMEM((NB, 128, WORDS), jnp.int32),
            pltpu.VMEM((2, 2, 8, 128), jnp.int32),
            (pltpu.SemaphoreType.DMA,) * 6,
        ],
        compiler_params=_SC_PARAMS,
    )
    return fb(idx3, table_i32)


def kernel(input, weight):
    idx3 = input.T.reshape(HIST, BTILES, 128).astype(jnp.int32)
    wp = jnp.pad(weight.T, ((0, 0), (0, 2 * PAD_PLANE - NUM_EMB)))
    out5 = _emb_call(idx3, wp)
    y = jax.lax.bitcast_convert_type(out5, jnp.bfloat16)
    return y.transpose(2, 4, 0, 1, 3, 5).reshape(BATCH, HIST, DIM)


# speedup vs baseline: 2.7312x; 1.5382x over previous
import jax
import jax.numpy as jnp
from jax import lax
from jax.experimental import pallas as pl
from jax.experimental.pallas import tpu as pltpu
from jax.experimental.pallas import tpu_sc as plsc

NUM_EMB = 1_000_000
DIM = 32
WORDS = DIM // 2
PAIRS = NUM_EMB // 2
BATCH = 16384
HIST = 50
NW = 32
NCOLS = NUM_EMB // 128
CTAIL = NUM_EMB - NCOLS * 128
BTILES = BATCH // 128
BT_W = BTILES // NW
NBLK = HIST * BT_W
NB = 4

_SC_PARAMS = pltpu.CompilerParams(
    use_tc_tiling_on_sc=False, needs_layout_passes=False)


def _worker_id():
    return lax.axis_index("s") * 2 + lax.axis_index("c")


_SC_TILED = pltpu.CompilerParams(
    use_tc_tiling_on_sc=True, needs_layout_passes=False)


def _format_body(wt_hbm, tr_hbm, buf, trbuf, tbuf, ttr, sin, sout):
    w = _worker_id()
    wsrc = wt_hbm.bitcast(jnp.int32)
    is_early = w < 4
    ncol = jnp.where(is_early, 245, 244)
    c_base = jnp.where(is_early, 245 * w, 980 + 244 * (w - 4))

    iot = lax.iota(jnp.int32, 16)
    idx_t = lax.shift_right_logical(iot, 3)
    idx_r = jnp.bitwise_and(iot, 7)

    def stage(b, slot):
        v0 = 128 * (c_base + b)
        for t in range(2):
            pltpu.async_copy(
                wsrc.at[pl.ds(8 * t, 8), pl.ds(v0, 128)],
                buf.at[slot, t], sin[slot])

    def drain_in(slot):
        d = pltpu.make_async_copy(
            wsrc.at[pl.ds(0, 8), pl.ds(0, 128)], buf.at[slot, 0], sin[slot])
        d.wait()
        d.wait()

    def drain_out(slot):
        pltpu.make_async_copy(
            trbuf.at[slot], tr_hbm.at[pl.ds(0, 2048)], sout[slot]).wait()

    def compute(slot):
        def grp(g, carry):
            for jj in range(16):
                j = g * 16 + jj
                colv = jnp.full((16,), 0, jnp.int32) + j
                vec = plsc.load_gather(buf.at[slot], [idx_t, idx_r, colv])
                trbuf[slot, pl.ds(16 * j, 16)] = vec
            return carry
        lax.fori_loop(0, 8, grp, 0)

    def writeback(b, slot):
        v0 = 128 * (c_base + b)
        pltpu.async_copy(
            trbuf.at[slot], tr_hbm.at[pl.ds(16 * v0, 2048)], sout[slot])

    stage(jnp.int32(0), 0)
    stage(jnp.int32(1), 1)

    def two_blocks(i, carry):
        for slot in range(2):
            b = 2 * i + slot
            drain_in(slot)

            @pl.when(i > 0)
            def _():
                drain_out(slot)

            compute(slot)
            writeback(b, slot)

            @pl.when(b + 2 < ncol)
            def _():
                stage(b + 2, slot)
        return carry

    lax.fori_loop(0, 122, two_blocks, 0)

    @pl.when(is_early)
    def _():
        b = jnp.int32(244)
        drain_in(0)
        drain_out(0)
        compute(0)
        writeback(b, 0)

    drain_out(0)
    drain_out(1)

    @pl.when(w == NW - 1)
    def _():
        v0 = NCOLS * 128
        for t in range(2):
            pltpu.sync_copy(
                wsrc.at[pl.ds(8 * t, 8), pl.ds(v0, CTAIL)], tbuf.at[t])
        for j in range(CTAIL):
            colv = jnp.full((16,), j, jnp.int32)
            vec = plsc.load_gather(tbuf, [idx_t, idx_r, colv])
            ttr[pl.ds(16 * j, 16)] = vec
        pltpu.sync_copy(ttr, tr_hbm.at[pl.ds(16 * v0, 16 * CTAIL)])


def _gather_body(idx_hbm, table_hbm, out_hbm, idx_v, rows_v, tb, sems):
    (sg0, sg1, sg2, sg3, sw0, sw1) = sems
    sgs = (sg0, sg1, sg2, sg3)
    sws = (sw0, sw1)
    w = _worker_id()

    pltpu.sync_copy(idx_hbm.at[:, pl.ds(BT_W * w, BT_W), :], idx_v)

    iot = lax.iota(jnp.int32, 16)

    def fire(k, slot):
        h = k // BT_W
        bb = lax.rem(k, BT_W)
        return pltpu.async_copy(
            table_hbm.at[idx_v.at[h, bb]], rows_v.at[slot], sgs[slot])

    def transpose_block(slot, tslot):
        for t in range(2):
            for r in range(8):
                colv = jnp.full((16,), 8 * t + r, jnp.int32)
                for g in range(8):
                    vec = plsc.load_gather(
                        rows_v.at[slot], [iot + 16 * g, colv])
                    tb[tslot, t, r, pl.ds(16 * g, 16)] = vec

    def writeback(k, tslot):
        h = k // BT_W
        bb = lax.rem(k, BT_W)
        bg = BT_W * w + bb
        pltpu.async_copy(tb.at[tslot, 0], out_hbm.at[h, 0, bg], sws[tslot])
        pltpu.async_copy(tb.at[tslot, 1], out_hbm.at[h, 1, bg], sws[tslot])

    def drain_gather(slot):
        pltpu.make_async_copy(
            table_hbm.at[idx_v.at[0, 0]], rows_v.at[slot], sgs[slot]).wait()

    def drain_wb(tslot):
        d = pltpu.make_async_copy(tb.at[tslot, 0], out_hbm.at[0, 0, 0], sws[tslot])
        d.wait()
        d.wait()

    for k in range(NB):
        fire(jnp.int32(k), k)

    def group(gi, carry):
        k0 = gi * NB
        for slot in range(NB):
            k = k0 + slot
            drain_gather(slot)
            tslot = slot % 2
            if slot < 2:
                @pl.when(gi > 0)
                def _():
                    drain_wb(tslot)
            else:
                drain_wb(tslot)
            transpose_block(slot, tslot)
            writeback(k, tslot)

            @pl.when(k + NB < NBLK)
            def _():
                fire(k + NB, slot)
        return carry

    lax.fori_loop(0, NBLK // NB, group, 0)
    drain_wb(0)
    drain_wb(1)


@jax.jit
def _emb_call(idx3, wt):
    mesh = plsc.VectorSubcoreMesh(core_axis_name="c", subcore_axis_name="s")
    fa = pl.kernel(
        _format_body,
        out_type=jax.ShapeDtypeStruct((NUM_EMB * WORDS,), jnp.int32),
        mesh=mesh,
        scratch_types=[
            pltpu.VMEM((2, 2, 8, 128), jnp.int32),
            pltpu.VMEM((2, 2048), jnp.int32),
            pltpu.VMEM((2, 8, CTAIL), jnp.int32),
            pltpu.VMEM((16 * CTAIL,), jnp.int32),
            (pltpu.SemaphoreType.DMA,) * 2,
            (pltpu.SemaphoreType.DMA,) * 2,
        ],
        compiler_params=_SC_TILED,
    )
    table_i32 = fa(wt).reshape(NUM_EMB, WORDS)
    fb = pl.kernel(
        _gather_body,
        out_type=jax.ShapeDtypeStruct((HIST, 2, BTILES, 8, 128), jnp.int32),
        mesh=mesh,
        scratch_types=[
            pltpu.VMEM((HIST, BT_W, 128), jnp.int32),
            pltpu.VMEM((NB, 128, WORDS), jnp.int32),
            pltpu.VMEM((2, 2, 8, 128), jnp.int32),
            (pltpu.SemaphoreType.DMA,) * 6,
        ],
        compiler_params=_SC_PARAMS,
    )
    return fb(idx3, table_i32)


def kernel(input, weight):
    idx3 = input.T.reshape(HIST, BTILES, 128).astype(jnp.int32)
    out5 = _emb_call(idx3, weight.T)
    y = jax.lax.bitcast_convert_type(out5, jnp.bfloat16)
    return y.transpose(2, 4, 0, 1, 3, 5).reshape(BATCH, HIST, DIM)
